# Initial kernel scaffold; baseline (speedup 1.0000x reference)
import jax
import jax.numpy as jnp
from jax.experimental import pallas as pl


def kernel(edge_vals, user_table, item_table, W0, W1, W2, edge_row, edge_col):
    raise NotImplementedError("write your pallas kernel here")


# speedup vs baseline: 5.0429x; 5.0429x over previous
import functools

import jax
import jax.numpy as jnp
from jax import lax
from jax.experimental import pallas as pl
from jax.experimental.pallas import tpu as pltpu
from jax.experimental.pallas import tpu_sc as plsc

D = 128
L = 16
CHUNK = 128
NC = 2
NS = 16
NW = NC * NS


def _spmm_body(N, n_chunks, emb_hbm, vals_hbm, row_hbm, col_hbm, out_hbm,
               col_v, row_v, vals_v, rows_v, acc, sem):
    c = lax.axis_index("c")
    s = lax.axis_index("s")
    w = s * NC + c

    rows_per_tile = 624
    r0 = s * rows_per_tile
    sub = (128, 128, 128, 128, 112)

    def _zero_row(i, carry):
        for k in range(D // L):
            rows_v[i, pl.ds(k * L, L)] = jnp.zeros((L,), jnp.float32)
        return carry
    lax.fori_loop(0, CHUNK, _zero_row, 0)
    off = 0
    for sz in sub:
        pltpu.sync_copy(rows_v.at[pl.ds(0, sz)],
                        acc.at[pl.ds(r0 + off, sz)])
        off += sz
    @pl.when(s == 0)
    def _zero_tail():
        pltpu.sync_copy(rows_v.at[pl.ds(0, 16)],
                        acc.at[pl.ds(NS * rows_per_tile, 16)])
    plsc.subcore_barrier()

    def _process(chunk_idx):
        base = chunk_idx * CHUNK
        pltpu.sync_copy(col_hbm.at[pl.ds(base, CHUNK)], col_v)
        pltpu.sync_copy(row_hbm.at[pl.ds(base, CHUNK)], row_v)
        pltpu.sync_copy(vals_hbm.at[pl.ds(base, CHUNK)], vals_v)
        pltpu.async_copy(emb_hbm.at[col_v], rows_v, sem).wait()

        def _scale(g, carry):
            grp = vals_v[pl.ds(g * L, L)]
            for e16 in range(L):
                sval = grp[e16]
                e = g * L + e16
                for k in range(D // L):
                    sl = pl.ds(k * L, L)
                    rows_v[e, sl] = rows_v[e, sl] * sval
            return carry
        lax.fori_loop(0, CHUNK // L, _scale, 0)
        pltpu.sync_copy(rows_v, acc.at[row_v], add=True)

    def _loop(j, carry):
        _process(j * NW + w)
        return carry
    lax.fori_loop(0, n_chunks // NW, _loop, 0)
    rem = n_chunks % NW
    if rem:
        @pl.when(w < rem)
        def _tail():
            _process((n_chunks // NW) * NW + w)

    plsc.subcore_barrier()

    pltpu.sync_copy(acc.at[pl.ds(r0, rows_per_tile)],
                    out_hbm.at[c, pl.ds(r0, rows_per_tile)])
    @pl.when(s == 0)
    def _write_tail():
        pltpu.sync_copy(acc.at[pl.ds(NS * rows_per_tile, 16)],
                        out_hbm.at[c, pl.ds(NS * rows_per_tile, 16)])


def _spmm(emb, edge_vals, edge_row, edge_col):
    N = emb.shape[0]
    E = edge_vals.shape[0]
    n_chunks = E // CHUNK
    mesh = plsc.VectorSubcoreMesh(core_axis_name="c", subcore_axis_name="s")
    f = pl.kernel(
        functools.partial(_spmm_body, N, n_chunks),
        out_type=jax.ShapeDtypeStruct((NC, N, D), jnp.float32),
        mesh=mesh,
        scratch_types=[
            pltpu.VMEM((CHUNK,), jnp.int32),
            pltpu.VMEM((CHUNK,), jnp.int32),
            pltpu.VMEM((CHUNK,), jnp.float32),
            pltpu.VMEM((CHUNK, D), jnp.float32),
            pltpu.VMEM_SHARED((N, D), jnp.float32),
            pltpu.SemaphoreType.DMA,
        ],
    )
    return f(emb, edge_vals, edge_row, edge_col)


def _dense_body(p_ref, w_ref, all_ref, emb_out_ref, all_out_ref):
    ssum = p_ref[0] + p_ref[1]
    h = jnp.dot(ssum, w_ref[...], preferred_element_type=jnp.float32)
    h = jnp.maximum(h, 0.0)
    nrm = jnp.sqrt(jnp.sum(h * h, axis=1, keepdims=True))
    h = h / jnp.maximum(nrm, 1e-12)
    emb_out_ref[...] = h
    all_out_ref[...] = all_ref[...] + h


def _dense(partials, W, all_emb):
    N = all_emb.shape[0]
    BLK = 1000
    return pl.pallas_call(
        _dense_body,
        grid=(N // BLK,),
        in_specs=[
            pl.BlockSpec((NC, BLK, D), lambda i: (0, i, 0)),
            pl.BlockSpec((D, D), lambda i: (0, 0)),
            pl.BlockSpec((BLK, D), lambda i: (i, 0)),
        ],
        out_specs=[
            pl.BlockSpec((BLK, D), lambda i: (i, 0)),
            pl.BlockSpec((BLK, D), lambda i: (i, 0)),
        ],
        out_shape=[
            jax.ShapeDtypeStruct((N, D), jnp.float32),
            jax.ShapeDtypeStruct((N, D), jnp.float32),
        ],
    )(partials, W, all_emb)


def kernel(edge_vals, user_table, item_table, W0, W1, W2, edge_row, edge_col):
    n_users = user_table.shape[0]
    emb = jnp.concatenate([user_table, item_table], axis=0)
    all_emb = emb
    for W in (W0, W1, W2):
        partials = _spmm(emb, edge_vals, edge_row, edge_col)
        emb, all_emb = _dense(partials, W, all_emb)
    return all_emb[:n_users], all_emb[n_users:]


# speedup vs baseline: 7.8786x; 1.5623x over previous
import functools

import jax
import jax.numpy as jnp
from jax import lax
from jax.experimental import pallas as pl
from jax.experimental.pallas import tpu as pltpu
from jax.experimental.pallas import tpu_sc as plsc

D = 128
L = 16
CHUNK = 80
NB = 3
NC = 2
NS = 16
NW = NC * NS


def _spmm_body(N, n_chunks, emb_hbm, vals_hbm, row_hbm, col_hbm, out_hbm,
               rows, cols, rowid, vals, acc, gsem, isem):
    cpt = n_chunks // NW

    c = lax.axis_index("c")
    s = lax.axis_index("s")
    w = s * NC + c
    cbase = w * cpt

    rows_per_tile = 624
    r0 = s * rows_per_tile

    def _zero_row(i, carry):
        for k in range(D // L):
            rows[0][i, pl.ds(k * L, L)] = jnp.zeros((L,), jnp.float32)
        return carry
    lax.fori_loop(0, CHUNK, _zero_row, 0)
    off = 0
    for sz in (80, 80, 80, 80, 80, 80, 80, 64):
        pltpu.sync_copy(rows[0].at[pl.ds(0, sz)],
                        acc.at[pl.ds(r0 + off, sz)])
        off += sz
    @pl.when(s == 0)
    def _zero_tail():
        pltpu.sync_copy(rows[0].at[pl.ds(0, 16)],
                        acc.at[pl.ds(NS * rows_per_tile, 16)])

    def _idx_copies(b, jj):
        base = (cbase + jj) * CHUNK
        return (
            (col_hbm.at[pl.ds(base, CHUNK)], cols[b]),
            (row_hbm.at[pl.ds(base, CHUNK)], rowid[b]),
            (vals_hbm.at[pl.ds(base, CHUNK)], vals[b]),
        )

    def _start_idx(b, jj):
        for src, dst in _idx_copies(b, jj):
            pltpu.async_copy(src, dst, isem[b])

    def _wait_idx(b, jj):
        for src, dst in _idx_copies(b, jj):
            pltpu.make_async_copy(src, dst, isem[b]).wait()

    def _start_gather(b):
        pltpu.async_copy(emb_hbm.at[cols[b]], rows[b], gsem[b])

    def _scale(b):
        def _grp(g, carry):
            grp = vals[b][pl.ds(g * L, L)]
            for e16 in range(L):
                sval = grp[e16]
                e = g * L + e16
                for k in range(D // L):
                    sl = pl.ds(k * L, L)
                    rows[b][e, sl] = rows[b][e, sl] * sval
            return carry
        lax.fori_loop(0, CHUNK // L, _grp, 0)

    _start_idx(0, 0)
    if cpt > 1:
        _start_idx(1, 1)
    _wait_idx(0, 0)
    _start_gather(0)

    def _visit(b, jj):
        pltpu.make_async_copy(emb_hbm.at[cols[b]], rows[b], gsem[b]).wait()
        _scale(b)
        @pl.when(jj + 2 < cpt)
        def _prefetch_idx():
            _start_idx((b + 2) % NB, jj + 2)
        @pl.when(jj + 1 < cpt)
        def _launch_gather():
            nb = (b + 1) % NB
            _wait_idx(nb, jj + 1)
            _start_gather(nb)
        pltpu.sync_copy(rows[b], acc.at[rowid[b]], add=True)

    def _ring(j, carry):
        for b in range(NB):
            jj = j * NB + b
            @pl.when(jj < cpt)
            def _v():
                _visit(b, jj)
        return carry
    lax.fori_loop(0, pl.cdiv(cpt, NB), _ring, 0)

    plsc.subcore_barrier()

    pltpu.sync_copy(acc.at[pl.ds(r0, rows_per_tile)],
                    out_hbm.at[c, pl.ds(r0, rows_per_tile)])
    @pl.when(s == 0)
    def _write_tail():
        pltpu.sync_copy(acc.at[pl.ds(NS * rows_per_tile, 16)],
                        out_hbm.at[c, pl.ds(NS * rows_per_tile, 16)])


def _spmm(emb, edge_vals, edge_row, edge_col):
    N = emb.shape[0]
    E = edge_vals.shape[0]
    n_chunks = E // CHUNK
    mesh = plsc.VectorSubcoreMesh(core_axis_name="c", subcore_axis_name="s")
    f = pl.kernel(
        functools.partial(_spmm_body, N, n_chunks),
        out_type=jax.ShapeDtypeStruct((NC, N, D), jnp.float32),
        mesh=mesh,
        scratch_types=[
            [pltpu.VMEM((CHUNK, D), jnp.float32) for _ in range(NB)],
            [pltpu.VMEM((CHUNK,), jnp.int32) for _ in range(NB)],
            [pltpu.VMEM((CHUNK,), jnp.int32) for _ in range(NB)],
            [pltpu.VMEM((CHUNK,), jnp.float32) for _ in range(NB)],
            pltpu.VMEM_SHARED((N, D), jnp.float32),
            [pltpu.SemaphoreType.DMA for _ in range(NB)],
            [pltpu.SemaphoreType.DMA for _ in range(NB)],
        ],
    )
    return f(emb, edge_vals, edge_row, edge_col)


def _dense_body(p_ref, w_ref, all_ref, emb_out_ref, all_out_ref):
    ssum = p_ref[0] + p_ref[1]
    h = jnp.dot(ssum, w_ref[...], preferred_element_type=jnp.float32)
    h = jnp.maximum(h, 0.0)
    nrm = jnp.sqrt(jnp.sum(h * h, axis=1, keepdims=True))
    h = h / jnp.maximum(nrm, 1e-12)
    emb_out_ref[...] = h
    all_out_ref[...] = all_ref[...] + h


def _dense(partials, W, all_emb):
    N = all_emb.shape[0]
    BLK = 1000
    return pl.pallas_call(
        _dense_body,
        grid=(N // BLK,),
        in_specs=[
            pl.BlockSpec((NC, BLK, D), lambda i: (0, i, 0)),
            pl.BlockSpec((D, D), lambda i: (0, 0)),
            pl.BlockSpec((BLK, D), lambda i: (i, 0)),
        ],
        out_specs=[
            pl.BlockSpec((BLK, D), lambda i: (i, 0)),
            pl.BlockSpec((BLK, D), lambda i: (i, 0)),
        ],
        out_shape=[
            jax.ShapeDtypeStruct((N, D), jnp.float32),
            jax.ShapeDtypeStruct((N, D), jnp.float32),
        ],
    )(partials, W, all_emb)


def kernel(edge_vals, user_table, item_table, W0, W1, W2, edge_row, edge_col):
    n_users = user_table.shape[0]
    emb = jnp.concatenate([user_table, item_table], axis=0)
    all_emb = emb
    for W in (W0, W1, W2):
        partials = _spmm(emb, edge_vals, edge_row, edge_col)
        emb, all_emb = _dense(partials, W, all_emb)
    return all_emb[:n_users], all_emb[n_users:]
